# baseline (speedup 1.0000x reference)
import functools

import jax
import jax.numpy as jnp
from jax import lax
from jax.experimental import pallas as pl
from jax.experimental.pallas import tpu as pltpu
from jax.experimental.pallas import tpu_sc as plsc

N = 10000
E = 320000
D = 128

NTILES = 16
NSL = 640
NPAD = NTILES * NSL
CH = 128
NST = 8
ER = E // CH
RPT = 160
EROWS = NTILES * RPT
NGRP = RPT // NST


def _sc_body(xp, rowp, colp, efr, out, agg_sh, nrm_sh,
             row3, col3, ef3, aw3, drow, dcol, dtmp, xb,
             gsem, ssem, rsem, csem):
    cid = lax.axis_index("c")
    sid = lax.axis_index("s")
    zero16 = jnp.zeros((16,), jnp.float32)
    one16 = jnp.ones((16,), jnp.float32)
    iota2 = 2 * lax.iota(jnp.int32, 16)
    scope = jax.named_scope

    def _zero_xb(i, c):
        for j in range(D // 16):
            xb[i, pl.ds(j * 16, 16)] = zero16
        return c
    lax.fori_loop(0, CH, _zero_xb, 0)
    for j in range(CH // 16):
        aw3[0, pl.ds(j * 16, 16)] = one16
    for k in range(NSL // CH):
        pltpu.sync_copy(xb.at[pl.ds(0, CH)],
                        agg_sh.at[pl.ds(sid * NSL + k * CH, CH)])
        pltpu.sync_copy(aw3.at[0], nrm_sh.at[pl.ds(sid * NSL + k * CH, CH)])
    plsc.subcore_barrier()

    def _deg_group(g, c):
        rbase = sid * RPT + g * NST

        @pl.when(g > 0)
        def _drain():
            for j in range(NST):
                pltpu.make_async_copy(
                    aw3.at[j], nrm_sh.at[col3.at[j]], rsem.at[j]).wait()
        pltpu.sync_copy(colp.at[pl.ds(rbase, NST)], col3)
        pltpu.sync_copy(efr.at[pl.ds(rbase, NST)], ef3)
        for j in range(NST):
            def _absj(v, cc, j=j):
                iidx = iota2 + (v * 32 + cid)
                ew = plsc.load_gather(ef3, [jnp.full((16,), j, jnp.int32),
                                            iidx])
                aw3[j, pl.ds(v * 16, 16)] = jnp.abs(ew)
                return cc
            lax.fori_loop(0, CH // 16, _absj, 0)
        for j in range(NST):
            pltpu.async_copy(aw3.at[j], nrm_sh.at[col3.at[j]], rsem.at[j],
                             add=True)
        return c
    with scope("p1_deg"):
        lax.fori_loop(0, NGRP, _deg_group, 0)
        for j in range(NST):
            pltpu.make_async_copy(aw3.at[j], nrm_sh.at[col3.at[j]],
                                  rsem.at[j]).wait()
        plsc.subcore_barrier()

    sl = pl.ds(sid * NSL, NSL)
    pltpu.sync_copy(nrm_sh.at[sl], dtmp)

    def _rsqrt_vreg(i, c):
        s = pl.ds(i * 16, 16)
        d = dtmp[s]
        ii = lax.bitcast_convert_type(d, jnp.int32)
        ii = jnp.int32(0x5F3759DF) - (ii >> 1)
        y = lax.bitcast_convert_type(ii, jnp.float32)
        for _ in range(3):
            y = y * (1.5 - 0.5 * d * y * y)
        dtmp[s] = y
        return c
    with scope("p2_rsqrt"):
        lax.fori_loop(0, NSL // 16, _rsqrt_vreg, 0)
        pltpu.sync_copy(dtmp, nrm_sh.at[sl])
        for k in range(NSL // CH):
            r0 = sid * NSL + k * CH
            pltpu.sync_copy(xp.at[pl.ds(r0, CH)], xb.at[pl.ds(0, CH)])

            def _init_vreg(v, c, k=k):
                dv = dtmp[pl.ds(k * CH + v * 16, 16)]
                dv = dv * dv
                for ln in range(16):
                    e = v * 16 + ln
                    cf = dv[ln]
                    for f in range(D // 16):
                        sf = pl.ds(f * 16, 16)
                        xb[e, sf] = xb[e, sf] * cf
                return c
            lax.fori_loop(0, CH // 16, _init_vreg, 0)
            pltpu.sync_copy(xb.at[pl.ds(0, CH)], agg_sh.at[pl.ds(r0, CH)])
        plsc.subcore_barrier()

    def _main_group(g, c):
        rbase = sid * RPT + g * NST

        @pl.when(g > 0)
        def _drain():
            for p in range(2):
                pltpu.make_async_copy(
                    xb.at[pl.ds(p * CH, CH)], agg_sh.at[col3.at[0]],
                    ssem.at[p]).wait()
        pltpu.sync_copy(rowp.at[pl.ds(rbase, NST)], row3)
        pltpu.sync_copy(colp.at[pl.ds(rbase, NST)], col3)
        pltpu.sync_copy(efr.at[pl.ds(rbase, NST)], ef3)
        for j in range(NST):
            pltpu.async_copy(nrm_sh.at[row3.at[j]], drow.at[j], rsem.at[j])
            pltpu.async_copy(nrm_sh.at[col3.at[j]], dcol.at[j], csem.at[j])
        pltpu.async_copy(xp.at[row3.at[0]], xb.at[pl.ds(0, CH)], gsem.at[0])
        for k in range(NST):
            p = k % 2
            q = 1 - p
            if k + 1 < NST:
                if k >= 1:
                    pltpu.make_async_copy(
                        xb.at[pl.ds(q * CH, CH)], agg_sh.at[col3.at[0]],
                        ssem.at[q]).wait()
                pltpu.async_copy(xp.at[row3.at[k + 1]],
                                 xb.at[pl.ds(q * CH, CH)], gsem.at[q])
            pltpu.make_async_copy(xp.at[row3.at[k]],
                                  xb.at[pl.ds(p * CH, CH)], gsem.at[p]).wait()
            pltpu.make_async_copy(nrm_sh.at[row3.at[k]], drow.at[k],
                                  rsem.at[k]).wait()
            pltpu.make_async_copy(nrm_sh.at[col3.at[k]], dcol.at[k],
                                  csem.at[k]).wait()

            def _vreg(v, cc, k=k, p=p):
                s = pl.ds(v * 16, 16)
                iidx = iota2 + (v * 32 + cid)
                ew = plsc.load_gather(ef3, [jnp.full((16,), k, jnp.int32),
                                            iidx])
                cf16 = drow[k, s] * jnp.abs(ew) * dcol[k, s]
                for ln in range(16):
                    e = p * CH + v * 16 + ln
                    cf = cf16[ln]
                    for f in range(D // 16):
                        sf = pl.ds(f * 16, 16)
                        xb[e, sf] = xb[e, sf] * cf
                return cc
            lax.fori_loop(0, CH // 16, _vreg, 0)
            pltpu.async_copy(xb.at[pl.ds(p * CH, CH)], agg_sh.at[col3.at[k]],
                             ssem.at[p], add=True)
        return c
    with scope("p3_main"):
        lax.fori_loop(0, NGRP, _main_group, 0)
        for p in range(2):
            pltpu.make_async_copy(xb.at[pl.ds(p * CH, CH)],
                                  agg_sh.at[col3.at[0]], ssem.at[p]).wait()
        plsc.subcore_barrier()

    with scope("p4_out"):
        for k in range(NSL // CH):
            r0 = sid * NSL + k * CH
            pltpu.sync_copy(agg_sh.at[pl.ds(r0, CH)], xb.at[pl.ds(0, CH)])
            pltpu.sync_copy(xb.at[pl.ds(0, CH)], out.at[cid, pl.ds(r0, CH)])


_sc_call = functools.partial(
    pl.kernel,
    out_type=jax.ShapeDtypeStruct((2, NPAD, D), jnp.float32),
    mesh=plsc.VectorSubcoreMesh(core_axis_name="c", subcore_axis_name="s"),
    compiler_params=pltpu.CompilerParams(needs_layout_passes=False),
    scratch_types=[
        pltpu.VMEM_SHARED((NPAD, D), jnp.float32),
        pltpu.VMEM_SHARED((NPAD,), jnp.float32),
        pltpu.VMEM((NST, CH), jnp.int32),
        pltpu.VMEM((NST, CH), jnp.int32),
        pltpu.VMEM((NST, 2 * CH), jnp.float32),
        pltpu.VMEM((NST, CH), jnp.float32),
        pltpu.VMEM((NST, CH), jnp.float32),
        pltpu.VMEM((NST, CH), jnp.float32),
        pltpu.VMEM((NSL,), jnp.float32),
        pltpu.VMEM((2 * CH, D), jnp.float32),
        pltpu.SemaphoreType.DMA((2,)),
        pltpu.SemaphoreType.DMA((2,)),
        pltpu.SemaphoreType.DMA((NST,)),
        pltpu.SemaphoreType.DMA((NST,)),
    ],
)(_sc_body)


def _mm_body(a_ref, w1_ref, w2_ref, b1_ref, b2_ref, o_ref):
    o_ref[:, :D] = (
        jnp.dot(a_ref[0], w1_ref[...], preferred_element_type=jnp.float32)
        + b1_ref[...]
    )
    o_ref[:, D:] = (
        jnp.dot(a_ref[1], w2_ref[...], preferred_element_type=jnp.float32)
        + b2_ref[...]
    )


_MB = 2000


def kernel(x, edge_index, edge_feat, W1, b1, W2, b2):
    idt = edge_index.dtype
    npadrows = RPT * NTILES - ER
    padi = (jnp.arange(npadrows * CH, dtype=idt) % N).reshape(npadrows, CH)
    rowp = jnp.concatenate([edge_index[0].reshape(ER, CH), padi])
    colp = jnp.concatenate([edge_index[1].reshape(ER, CH), padi])
    efr = jnp.concatenate([
        edge_feat.reshape(ER, 2 * CH),
        jnp.zeros((npadrows, 2 * CH), jnp.float32),
    ])
    xp = jnp.pad(x, ((0, NPAD - N), (0, 0)))

    agg = _sc_call(xp, rowp, colp, efr)

    out = pl.pallas_call(
        _mm_body,
        grid=(N // _MB,),
        in_specs=[
            pl.BlockSpec((2, _MB, D), lambda i: (0, i, 0)),
            pl.BlockSpec((D, D), lambda i: (0, 0)),
            pl.BlockSpec((D, D), lambda i: (0, 0)),
            pl.BlockSpec((1, D), lambda i: (0, 0)),
            pl.BlockSpec((1, D), lambda i: (0, 0)),
        ],
        out_specs=pl.BlockSpec((_MB, 2 * D), lambda i: (i, 0)),
        out_shape=jax.ShapeDtypeStruct((N, 2 * D), jnp.float32),
    )(agg, W1, W2, b1.reshape(1, D), b2.reshape(1, D))
    return out


# speedup vs baseline: 1.4643x; 1.4643x over previous
import functools

import jax
import jax.numpy as jnp
from jax import lax
from jax.experimental import pallas as pl
from jax.experimental.pallas import tpu as pltpu
from jax.experimental.pallas import tpu_sc as plsc

N = 10000
E = 320000
D = 128

NTILES = 16
NSL = 640
NPAD = NTILES * NSL
CH = 128
NST = 8
GRP = NST * CH
EPT = E // NTILES
NGRP = 19
TAIL = EPT - NGRP * GRP
TOFF = (0, CH, 2 * CH, 3 * CH, TAIL - CH)
TMSK = (0, 0, 0, 0, 6)
WIN = GRP + CH
TWIN = 640


def _sc_body(xp, eidx, efT, out, agg_sh, nrm_sh,
             ri2d, row3, col3, efw, aw3, drow, dcol, dtmp, xb,
             gsem, ssem, rsem, csem):
    cid = lax.axis_index("c")
    sid = lax.axis_index("s")
    zero16 = jnp.zeros((16,), jnp.float32)
    one16 = jnp.ones((16,), jnp.float32)
    iota16 = lax.iota(jnp.int32, 16)
    scope = jax.named_scope

    def _repack(off, j):
        for v in range(CH // 16):
            row3[j, pl.ds(v * 16, 16)] = ri2d[0, pl.ds(off + v * 16, 16)]
            col3[j, pl.ds(v * 16, 16)] = ri2d[1, pl.ds(off + v * 16, 16)]

    def _repack_col(off, j):
        for v in range(CH // 16):
            col3[j, pl.ds(v * 16, 16)] = ri2d[1, pl.ds(off + v * 16, 16)]

    def _ew16(woff, v):
        s = pl.ds(woff + v * 16, 16)
        return jnp.abs(jnp.where(cid == 0, efw[0, s], efw[1, s]))

    def _zero_xb(i, c):
        for j in range(D // 16):
            xb[i, pl.ds(j * 16, 16)] = zero16
        return c
    lax.fori_loop(0, CH, _zero_xb, 0)
    for j in range(CH // 16):
        aw3[0, pl.ds(j * 16, 16)] = one16
    for k in range(NSL // CH):
        pltpu.sync_copy(xb.at[pl.ds(0, CH)],
                        agg_sh.at[pl.ds(sid * NSL + k * CH, CH)])
        pltpu.sync_copy(aw3.at[0], nrm_sh.at[pl.ds(sid * NSL + k * CH, CH)])
    plsc.subcore_barrier()

    def _deg_sub(j, off, mv):
        def _absj(v, cc):
            aw3[j, pl.ds(v * 16, 16)] = _ew16(off, v)
            return cc
        for v in range(mv):
            aw3[j, pl.ds(v * 16, 16)] = zero16
        lax.fori_loop(mv, CH // 16, _absj, 0)

    def _deg_group(g, c):
        base = sid * EPT + g * GRP

        @pl.when(g > 0)
        def _drain():
            for j in range(NST):
                pltpu.make_async_copy(
                    aw3.at[j], nrm_sh.at[col3.at[j]], rsem.at[j]).wait()
        abase = (base // CH) * CH
        aoff = base - abase
        pltpu.sync_copy(eidx.at[:, pl.ds(abase, WIN)], ri2d)
        pltpu.sync_copy(efT.at[:, pl.ds(abase, WIN)], efw)
        for j in range(NST):
            _repack_col(aoff + j * CH, j)
            _deg_sub(j, aoff + j * CH, 0)
        for j in range(NST):
            pltpu.async_copy(aw3.at[j], nrm_sh.at[col3.at[j]], rsem.at[j],
                             add=True)
        return c
    with scope("p1_deg"):
        lax.fori_loop(0, NGRP, _deg_group, 0)
        for j in range(NST):
            pltpu.make_async_copy(aw3.at[j], nrm_sh.at[col3.at[j]],
                                  rsem.at[j]).wait()
        tbase = sid * EPT + NGRP * GRP
        tabase = (tbase // CH) * CH
        taoff = tbase - tabase
        pltpu.sync_copy(eidx.at[:, pl.ds(tabase, TWIN)],
                        ri2d.at[:, pl.ds(0, TWIN)])
        pltpu.sync_copy(efT.at[:, pl.ds(tabase, TWIN)],
                        efw.at[:, pl.ds(0, TWIN)])
        for j in range(5):
            _repack_col(taoff + TOFF[j], j)
            _deg_sub(j, taoff + TOFF[j], TMSK[j])
        for j in range(5):
            pltpu.async_copy(aw3.at[j], nrm_sh.at[col3.at[j]], rsem.at[j],
                             add=True)
        for j in range(5):
            pltpu.make_async_copy(aw3.at[j], nrm_sh.at[col3.at[j]],
                                  rsem.at[j]).wait()
        plsc.subcore_barrier()

    sl = pl.ds(sid * NSL, NSL)
    pltpu.sync_copy(nrm_sh.at[sl], dtmp)

    def _rsqrt_vreg(i, c):
        s = pl.ds(i * 16, 16)
        d = dtmp[s]
        ii = lax.bitcast_convert_type(d, jnp.int32)
        ii = jnp.int32(0x5F3759DF) - (ii >> 1)
        y = lax.bitcast_convert_type(ii, jnp.float32)
        for _ in range(3):
            y = y * (1.5 - 0.5 * d * y * y)
        dtmp[s] = y
        return c
    with scope("p2_rsqrt"):
        lax.fori_loop(0, NSL // 16, _rsqrt_vreg, 0)
        pltpu.sync_copy(dtmp, nrm_sh.at[sl])
        for k in range(NSL // CH):
            r0 = sid * NSL + k * CH
            pltpu.sync_copy(xp.at[pl.ds(r0, CH)], xb.at[pl.ds(0, CH)])

            def _init_vreg(v, c, k=k):
                dv = dtmp[pl.ds(k * CH + v * 16, 16)]
                dv = dv * dv
                for ln in range(16):
                    e = v * 16 + ln
                    cf = dv[ln]
                    for f in range(D // 16):
                        sf = pl.ds(f * 16, 16)
                        xb[e, sf] = xb[e, sf] * cf
                return c
            lax.fori_loop(0, CH // 16, _init_vreg, 0)
            pltpu.sync_copy(xb.at[pl.ds(0, CH)], agg_sh.at[pl.ds(r0, CH)])
        plsc.subcore_barrier()

    def _scale_sub(j, eoff, mv, p):
        pltpu.make_async_copy(nrm_sh.at[row3.at[j]],
                              drow.at[j], rsem.at[j]).wait()
        pltpu.make_async_copy(nrm_sh.at[col3.at[j]],
                              dcol.at[j], csem.at[j]).wait()

        def _zero_vreg(v, cc):
            for ln in range(16):
                e = p * CH + v * 16 + ln
                for f in range(D // 16):
                    xb[e, pl.ds(f * 16, 16)] = zero16
            return cc
        if mv:
            lax.fori_loop(0, mv, _zero_vreg, 0)

        def _vreg(v, cc):
            s = pl.ds(v * 16, 16)
            cf16 = drow[j, s] * _ew16(eoff, v) * dcol[j, s]
            for ln in range(16):
                e = p * CH + v * 16 + ln
                cf = cf16[ln]
                for f in range(D // 16):
                    sf = pl.ds(f * 16, 16)
                    xb[e, sf] = xb[e, sf] * cf
            return cc
        lax.fori_loop(mv, CH // 16, _vreg, 0)

    def _main_group(g, c):
        base = sid * EPT + g * GRP

        @pl.when(g > 0)
        def _drain():
            for p in range(2):
                pltpu.make_async_copy(
                    xb.at[pl.ds(p * CH, CH)], agg_sh.at[col3.at[0]],
                    ssem.at[p]).wait()
        abase = (base // CH) * CH
        aoff = base - abase
        pltpu.sync_copy(eidx.at[:, pl.ds(abase, WIN)], ri2d)
        pltpu.sync_copy(efT.at[:, pl.ds(abase, WIN)], efw)
        for j in range(NST):
            _repack(aoff + j * CH, j)
            pltpu.async_copy(nrm_sh.at[row3.at[j]], drow.at[j], rsem.at[j])
            pltpu.async_copy(nrm_sh.at[col3.at[j]], dcol.at[j], csem.at[j])
        pltpu.async_copy(xp.at[row3.at[0]],
                         xb.at[pl.ds(0, CH)], gsem.at[0])
        for k in range(NST):
            p = k % 2
            q = 1 - p
            if k + 1 < NST:
                if k >= 1:
                    pltpu.make_async_copy(
                        xb.at[pl.ds(q * CH, CH)], agg_sh.at[col3.at[0]],
                        ssem.at[q]).wait()
                pltpu.async_copy(
                    xp.at[row3.at[k + 1]],
                    xb.at[pl.ds(q * CH, CH)], gsem.at[q])
            pltpu.make_async_copy(xp.at[row3.at[k]],
                                  xb.at[pl.ds(p * CH, CH)], gsem.at[p]).wait()
            _scale_sub(k, aoff + k * CH, 0, p)
            pltpu.async_copy(xb.at[pl.ds(p * CH, CH)], agg_sh.at[col3.at[k]],
                             ssem.at[p], add=True)
        return c
    with scope("p3_main"):
        lax.fori_loop(0, NGRP, _main_group, 0)
        for p in range(2):
            pltpu.make_async_copy(xb.at[pl.ds(p * CH, CH)],
                                  agg_sh.at[col3.at[0]], ssem.at[p]).wait()
        tbase = sid * EPT + NGRP * GRP
        tabase = (tbase // CH) * CH
        taoff = tbase - tabase
        pltpu.sync_copy(eidx.at[:, pl.ds(tabase, TWIN)],
                        ri2d.at[:, pl.ds(0, TWIN)])
        pltpu.sync_copy(efT.at[:, pl.ds(tabase, TWIN)],
                        efw.at[:, pl.ds(0, TWIN)])
        for j in range(5):
            _repack(taoff + TOFF[j], j)
            pltpu.async_copy(nrm_sh.at[row3.at[j]], drow.at[j], rsem.at[j])
            pltpu.async_copy(nrm_sh.at[col3.at[j]], dcol.at[j], csem.at[j])
        pltpu.async_copy(xp.at[row3.at[0]],
                         xb.at[pl.ds(0, CH)], gsem.at[0])
        for k in range(5):
            p = k % 2
            q = 1 - p
            if k + 1 < 5:
                if k >= 1:
                    pltpu.make_async_copy(
                        xb.at[pl.ds(q * CH, CH)], agg_sh.at[col3.at[0]],
                        ssem.at[q]).wait()
                pltpu.async_copy(
                    xp.at[row3.at[k + 1]],
                    xb.at[pl.ds(q * CH, CH)], gsem.at[q])
            pltpu.make_async_copy(
                xp.at[row3.at[k]],
                xb.at[pl.ds(p * CH, CH)], gsem.at[p]).wait()
            _scale_sub(k, taoff + TOFF[k], TMSK[k], p)
            pltpu.async_copy(xb.at[pl.ds(p * CH, CH)], agg_sh.at[col3.at[k]],
                             ssem.at[p], add=True)
        for p in range(2):
            pltpu.make_async_copy(xb.at[pl.ds(p * CH, CH)],
                                  agg_sh.at[col3.at[0]], ssem.at[p]).wait()
        plsc.subcore_barrier()

    with scope("p4_out"):
        for k in range(NSL // CH):
            r0 = sid * NSL + k * CH
            pltpu.sync_copy(agg_sh.at[pl.ds(r0, CH)], xb.at[pl.ds(0, CH)])
            pltpu.sync_copy(xb.at[pl.ds(0, CH)], out.at[cid, pl.ds(r0, CH)])


_sc_call = functools.partial(
    pl.kernel,
    out_type=jax.ShapeDtypeStruct((2, NPAD, D), jnp.float32),
    mesh=plsc.VectorSubcoreMesh(core_axis_name="c", subcore_axis_name="s"),
    compiler_params=pltpu.CompilerParams(needs_layout_passes=False),
    scratch_types=[
        pltpu.VMEM_SHARED((NPAD, D), jnp.float32),
        pltpu.VMEM_SHARED((NPAD,), jnp.float32),
        pltpu.VMEM((2, WIN), jnp.int32),
        pltpu.VMEM((NST, CH), jnp.int32),
        pltpu.VMEM((NST, CH), jnp.int32),
        pltpu.VMEM((2, WIN), jnp.float32),
        pltpu.VMEM((NST, CH), jnp.float32),
        pltpu.VMEM((NST, CH), jnp.float32),
        pltpu.VMEM((NST, CH), jnp.float32),
        pltpu.VMEM((NSL,), jnp.float32),
        pltpu.VMEM((2 * CH, D), jnp.float32),
        pltpu.SemaphoreType.DMA((2,)),
        pltpu.SemaphoreType.DMA((2,)),
        pltpu.SemaphoreType.DMA((NST,)),
        pltpu.SemaphoreType.DMA((NST,)),
    ],
)(_sc_body)


def _mm_body(a_ref, w1_ref, w2_ref, b1_ref, b2_ref, o_ref):
    o_ref[:, :D] = (
        jnp.dot(a_ref[0], w1_ref[...], preferred_element_type=jnp.float32)
        + b1_ref[...]
    )
    o_ref[:, D:] = (
        jnp.dot(a_ref[1], w2_ref[...], preferred_element_type=jnp.float32)
        + b2_ref[...]
    )


_MB = 2000


def kernel(x, edge_index, edge_feat, W1, b1, W2, b2):
    xp = jnp.pad(x, ((0, NPAD - N), (0, 0)))
    agg = _sc_call(xp, edge_index, edge_feat.T)

    out = pl.pallas_call(
        _mm_body,
        grid=(N // _MB,),
        in_specs=[
            pl.BlockSpec((2, _MB, D), lambda i: (0, i, 0)),
            pl.BlockSpec((D, D), lambda i: (0, 0)),
            pl.BlockSpec((D, D), lambda i: (0, 0)),
            pl.BlockSpec((1, D), lambda i: (0, 0)),
            pl.BlockSpec((1, D), lambda i: (0, 0)),
        ],
        out_specs=pl.BlockSpec((_MB, 2 * D), lambda i: (i, 0)),
        out_shape=jax.ShapeDtypeStruct((N, 2 * D), jnp.float32),
    )(agg, W1, W2, b1.reshape(1, D), b2.reshape(1, D))
    return out
